# baseline (speedup 1.0000x reference)
import jax
import jax.numpy as jnp
from jax.experimental import pallas as pl


def _mlp_apply(layers, x):
    for i, l in enumerate(layers):
        x = x @ l["W"]
        if l["b"] is not None:
            x = x + l["b"]
        if i < len(layers) - 1:
            x = jax.nn.relu(x)
    return x


def _in_apply(p, x, src, dst, e, n, mask=None):
    xe = jnp.concatenate([x[src], x[dst], e], axis=1)
    e_out = _mlp_apply(p["phi_e"], xe)
    msg = e_out if mask is None else e_out * mask
    agg = jax.ops.segment_sum(msg, dst, num_segments=n)
    x_out = _mlp_apply(p["phi_x"], jnp.concatenate([x, agg], axis=1))
    return x_out, e_out


def _identity_kernel(x_ref, o_ref):
    o_ref[...] = x_ref[...]


def _pallas_identity(x):
    return pl.pallas_call(
        _identity_kernel,
        out_shape=jax.ShapeDtypeStruct(x.shape, x.dtype),
    )(x)


def kernel(x, edge_attr, params, edge_index):
    n = x.shape[0]
    src, dst = edge_index[0], edge_index[1]
    h_ec = jax.nn.relu(_mlp_apply(params["ec_node_enc"], x))
    e_ec = jax.nn.relu(_mlp_apply(params["ec_edge_enc"], edge_attr))
    _, e_ec2 = _in_apply(params["ec_in"], h_ec, src, dst, e_ec, n)
    edge_weights = jax.nn.sigmoid(_mlp_apply(params["ec_w"], e_ec2))
    mask = (edge_weights > 0.5).astype(jnp.float32)
    h = jax.nn.relu(_mlp_apply(params["hc_node_enc"], x))
    e = jax.nn.relu(_mlp_apply(params["hc_edge_enc"], edge_attr))
    edge_attrs = [e]
    for layer in params["hc_in"]:
        h_new, e_new = _in_apply(layer, h, src, dst, e, n, mask=mask)
        h = h + h_new
        e = e + e_new
        edge_attrs.append(e)
    beta = jax.nn.sigmoid(_mlp_apply(params["p_beta"], h)) + 1e-8
    H = _mlp_apply(params["p_cluster"], h)
    e_cat = jnp.concatenate(edge_attrs, axis=1)
    P, _ = _in_apply(params["p_track"], h, src, dst, e_cat, n, mask=mask)
    return edge_weights, _pallas_identity(H), beta, P
